# baseline (speedup 1.0000x reference)
import functools

import jax
import jax.numpy as jnp
from jax import lax
from jax.experimental import pallas as pl
from jax.experimental.pallas import tpu as pltpu
from jax.experimental.pallas import tpu_sc as plsc

D_MODEL_K = 64
MAX_LEN_K = 13941
ROWS_PAD = 13952

NC = 2
NS = 16
NW = NC * NS


def _blur_body(a0, a1, a2, out):
    out[...] = 0.5 * a0[...] + a1[...] + 0.5 * a2[...]


def _blur(a0, a1, a2):
    shp = jax.ShapeDtypeStruct(a0.shape, jnp.float32)
    return pl.pallas_call(_blur_body, out_shape=shp)(a0, a1, a2)


def _make_gather(n_b, n_s):
    s_per_w = n_b // NW
    mesh = plsc.VectorSubcoreMesh(core_axis_name="c", subcore_axis_name="s")

    @functools.partial(
        pl.kernel,
        mesh=mesh,
        out_type=jax.ShapeDtypeStruct((n_b, n_s, D_MODEL_K), jnp.float32),
        scratch_types=[
            pltpu.VMEM((n_s,), jnp.int32),
            pltpu.VMEM((n_s, 128), jnp.float32),
            pltpu.VMEM((n_s, D_MODEL_K), jnp.float32),
            pltpu.SemaphoreType.DMA,
        ],
    )
    def gather_k(table_hbm, idx_hbm, out_hbm, idx_v, rows_v, cpt_v, sem):
        wid = lax.axis_index("s") * NC + lax.axis_index("c")
        base = wid * s_per_w

        def chunk_body(i, carry):
            s = base + i
            pltpu.sync_copy(idx_hbm.at[pl.ds(s * n_s, n_s)], idx_v)
            pltpu.async_copy(table_hbm.at[idx_v], rows_v, sem).wait()

            def compact_row(r, c2):
                for c in range(D_MODEL_K // 16):
                    cpt_v[r, pl.ds(c * 16, 16)] = rows_v[r, pl.ds(c * 16, 16)]
                return c2

            lax.fori_loop(0, n_s, compact_row, 0)
            pltpu.sync_copy(cpt_v, out_hbm.at[s])
            return carry

        lax.fori_loop(0, s_per_w, chunk_body, 0)

    return gather_k


def kernel(pos, table):
    t = table.astype(jnp.float32)
    a0 = jnp.concatenate([t[0:1], t[: MAX_LEN_K - 1]], axis=0)
    a1 = t[:MAX_LEN_K]
    a2 = t[1 : MAX_LEN_K + 1]
    pad = ROWS_PAD - MAX_LEN_K
    a0, a1, a2 = (
        jnp.pad(x, ((0, pad), (0, 0))).reshape(ROWS_PAD * D_MODEL_K // 128, 128)
        for x in (a0, a1, a2)
    )
    blurred = _blur(a0, a1, a2).reshape(ROWS_PAD, D_MODEL_K)
    blurred = jnp.pad(blurred, ((0, 0), (0, 128 - D_MODEL_K)))

    b, s = pos.shape
    idx = pos.reshape(-1).astype(jnp.int32)
    return _make_gather(b, s)(blurred, idx)


# speedup vs baseline: 1.3354x; 1.3354x over previous
import functools

import jax
import jax.numpy as jnp
from jax import lax
from jax.experimental import pallas as pl
from jax.experimental.pallas import tpu as pltpu
from jax.experimental.pallas import tpu_sc as plsc

D_MODEL_K = 64
MAX_LEN_K = 13941
ROWS_PAD = 13952

NC = 2
NS = 16
NW = NC * NS


def _blur_body(a0, a1, a2, out):
    out[...] = 0.5 * a0[...] + a1[...] + 0.5 * a2[...]


def _blur(a0, a1, a2):
    shp = jax.ShapeDtypeStruct(a0.shape, jnp.float32)
    return pl.pallas_call(_blur_body, out_shape=shp)(a0, a1, a2)


def _make_gather(n_b, n_s):
    s_per_w = n_b // NW
    mesh = plsc.VectorSubcoreMesh(core_axis_name="c", subcore_axis_name="s")

    @functools.partial(
        pl.kernel,
        mesh=mesh,
        out_type=jax.ShapeDtypeStruct((n_b, n_s, D_MODEL_K), jnp.float32),
        scratch_types=[
            pltpu.VMEM((s_per_w * n_s,), jnp.int32),
            pltpu.VMEM((2, n_s, 128), jnp.float32),
            pltpu.VMEM((2, n_s, D_MODEL_K), jnp.float32),
            pltpu.SemaphoreType.DMA,
            pltpu.SemaphoreType.DMA,
            pltpu.SemaphoreType.DMA,
            pltpu.SemaphoreType.DMA,
        ],
    )
    def gather_k(table_hbm, idx_hbm, out_hbm, idx_v, rows_v, cpt_v, g0, g1, w0, w1):
        wid = lax.axis_index("s") * NC + lax.axis_index("c")
        base = wid * s_per_w
        gsem = (g0, g1)
        wsem = (w0, w1)

        def start_gather(i, b):
            pltpu.async_copy(
                table_hbm.at[idx_v.at[pl.ds(i * n_s, n_s)]], rows_v.at[b], gsem[b]
            )

        def wait_gather(b):
            pltpu.make_async_copy(
                table_hbm.at[pl.ds(0, n_s)], rows_v.at[b], gsem[b]
            ).wait()

        def compact(b):
            def body(t, c):
                r0 = t * 8
                for rr in range(8):
                    for c4 in range(D_MODEL_K // 16):
                        cpt_v[b, r0 + rr, pl.ds(c4 * 16, 16)] = rows_v[
                            b, r0 + rr, pl.ds(c4 * 16, 16)
                        ]
                return c

            lax.fori_loop(0, n_s // 8, body, 0, unroll=False)

        pltpu.sync_copy(idx_hbm.at[pl.ds(base * n_s, s_per_w * n_s)], idx_v)
        start_gather(0, 0)

        def slot(i, k, j):
            if j == 0:
                start_gather(i + 1, 1)
            else:
                @pl.when(k < s_per_w // 2 - 1)
                def _():
                    start_gather(i + 1, 0)
            wait_gather(j)

            @pl.when(k >= 1)
            def _():
                pltpu.make_async_copy(
                    cpt_v.at[j], out_hbm.at[base + i - 2], wsem[j]
                ).wait()

            compact(j)
            pltpu.async_copy(cpt_v.at[j], out_hbm.at[base + i], wsem[j])

        def outer(k, c):
            slot(2 * k, k, 0)
            slot(2 * k + 1, k, 1)
            return c

        lax.fori_loop(0, s_per_w // 2, outer, 0, unroll=False)

        for j in range(2):
            pltpu.make_async_copy(
                cpt_v.at[j], out_hbm.at[base + s_per_w - 2 + j], wsem[j]
            ).wait()

    return gather_k


def kernel(pos, table):
    t = table.astype(jnp.float32)
    a0 = jnp.concatenate([t[0:1], t[: MAX_LEN_K - 1]], axis=0)
    a1 = t[:MAX_LEN_K]
    a2 = t[1 : MAX_LEN_K + 1]
    pad = ROWS_PAD - MAX_LEN_K
    a0, a1, a2 = (
        jnp.pad(x, ((0, pad), (0, 0))).reshape(ROWS_PAD * D_MODEL_K // 128, 128)
        for x in (a0, a1, a2)
    )
    blurred = _blur(a0, a1, a2).reshape(ROWS_PAD, D_MODEL_K)
    blurred = jnp.pad(blurred, ((0, 0), (0, 128 - D_MODEL_K)))

    b, s = pos.shape
    idx = pos.reshape(-1).astype(jnp.int32)
    return _make_gather(b, s)(blurred, idx)
